# baseline (speedup 1.0000x reference)
import functools

import jax
import jax.numpy as jnp
from jax import lax
from jax.experimental import pallas as pl
from jax.experimental.pallas import tpu as pltpu
from jax.experimental.pallas import tpu_sc as plsc

_ROW = 16384
_NW = 32
_ROWS_PER_W = 4096 // _NW


def _sc_body(in_hbm, out_hbm, in_v, out_v):
    wid = lax.axis_index("s") * 2 + lax.axis_index("c")
    iota2 = lax.iota(jnp.int32, 16) * 2

    def row_step(t, carry):
        r = wid * _ROWS_PER_W + t
        bc = r >> 7
        hr = r & 127
        i = hr & 1
        h = hr >> 1
        pltpu.sync_copy(in_hbm.at[r], in_v)

        for jk in range(4):
            j, k = jk >> 1, jk & 1

            def w_step(w, c2, jk=jk, j=j, k=k):
                base = 256 * w + 128 * j + k
                off = jk * 4096 + w * 64
                for z16 in range(4):
                    idx = iota2 + (base + 32 * z16)
                    v = plsc.load_gather(in_v, [idx])
                    out_v[pl.ds(off + z16 * 16, 16)] = v
                return c2

            lax.fori_loop(0, 64, w_step, 0, unroll=4)

        orow_base = bc * 512 + i * 256 + h
        for jk in range(4):
            pltpu.sync_copy(
                out_v.at[pl.ds(jk * 4096, 4096)],
                out_hbm.at[orow_base + jk * 64],
            )
        return carry

    lax.fori_loop(0, _ROWS_PER_W, row_step, 0)


def kernel(x):
    B, C, H, W, Z = x.shape
    BC = B * C
    xf = x.reshape(BC * H, W * Z)
    mesh = plsc.VectorSubcoreMesh(core_axis_name="c", subcore_axis_name="s")
    run = functools.partial(
        pl.kernel,
        mesh=mesh,
        out_type=jax.ShapeDtypeStruct((BC * 8 * (H // 2), (W // 2) * (Z // 2)), x.dtype),
        scratch_types=[
            pltpu.VMEM((_ROW,), jnp.float32),
            pltpu.VMEM((_ROW,), jnp.float32),
        ],
        compiler_params=pltpu.CompilerParams(needs_layout_passes=False),
    )(_sc_body)
    out = run(xf)
    return out.reshape(B, C * 8, H // 2, W // 2, Z // 2)


# speedup vs baseline: 1.4073x; 1.4073x over previous
import functools

import jax
import jax.numpy as jnp
from jax import lax
from jax.experimental import pallas as pl
from jax.experimental.pallas import tpu as pltpu
from jax.experimental.pallas import tpu_sc as plsc

_L = 16384
_OC = 4096
_NW = 32
_RPW = 4096 // _NW


def _sc_body(in_hbm, out_hbm, iv0, iv1, ov0, ov1, si0, si1, so0, so1):
    wid = lax.axis_index("s") * 2 + lax.axis_index("c")
    iota2 = lax.iota(jnp.int32, 16) * 2
    ivs = (iv0, iv1)
    ovs = (ov0, ov1)
    sis = (si0, si1)
    sos = (so0, so1)

    def in_cp(t, b):
        r = wid * _RPW + t
        return pltpu.make_async_copy(
            in_hbm.at[pl.ds(r * _L, _L)], ivs[b], sis[b]
        )

    def out_cp(t, b, jk):
        r = wid * _RPW + t
        bc = r >> 7
        hr = r & 127
        orow = bc * 512 + (hr & 1) * 256 + jk * 64 + (hr >> 1)
        return pltpu.make_async_copy(
            ovs[b].at[pl.ds(jk * _OC, _OC)],
            out_hbm.at[pl.ds(orow * _OC, _OC)],
            sos[b],
        )

    def compute(b):
        src = ivs[b]
        dst = ovs[b]
        for jk in range(4):
            j, k = jk >> 1, jk & 1
            c0 = 128 * j + k

            def w_step(w, idxs, jk=jk, dst=dst, src=src):
                off = jk * _OC + w * 64
                nxt = []
                for z16 in range(4):
                    v = plsc.load_gather(src, [idxs[z16]])
                    dst[pl.ds(off + z16 * 16, 16)] = v
                    nxt.append(idxs[z16] + 256)
                return tuple(nxt)

            init = tuple(iota2 + (c0 + 32 * z16) for z16 in range(4))
            lax.fori_loop(0, 64, w_step, init, unroll=2)

    def full_step(t, b):
        in_cp(t, b).wait()
        for jk in range(4):
            out_cp(t, b, jk).wait()
        compute(b)
        in_cp(t + 2, b).start()
        for jk in range(4):
            out_cp(t, b, jk).start()

    in_cp(0, 0).start()
    in_cp(1, 1).start()
    for b in range(2):
        in_cp(b, b).wait()
        compute(b)
        in_cp(b + 2, b).start()
        for jk in range(4):
            out_cp(b, b, jk).start()

    def loop(tb, carry):
        full_step(tb, 0)
        full_step(tb + 1, 1)
        return carry

    lax.fori_loop(1, _RPW // 2 - 1, lambda u, c: loop(2 * u, c), 0)

    for b in range(2):
        t = _RPW - 2 + b
        in_cp(t, b).wait()
        for jk in range(4):
            out_cp(t, b, jk).wait()
        compute(b)
        for jk in range(4):
            out_cp(t, b, jk).start()
    for b in range(2):
        for jk in range(4):
            out_cp(0, b, jk).wait()


def kernel(x):
    B, C, H, W, Z = x.shape
    BC = B * C
    n = BC * H * W * Z
    xf = x.reshape(n)
    mesh = plsc.VectorSubcoreMesh(core_axis_name="c", subcore_axis_name="s")
    run = functools.partial(
        pl.kernel,
        mesh=mesh,
        out_type=jax.ShapeDtypeStruct((n,), x.dtype),
        scratch_types=[
            pltpu.VMEM((_L,), jnp.float32),
            pltpu.VMEM((_L,), jnp.float32),
            pltpu.VMEM((_L,), jnp.float32),
            pltpu.VMEM((_L,), jnp.float32),
            pltpu.SemaphoreType.DMA,
            pltpu.SemaphoreType.DMA,
            pltpu.SemaphoreType.DMA,
            pltpu.SemaphoreType.DMA,
        ],
        compiler_params=pltpu.CompilerParams(needs_layout_passes=False),
    )(_sc_body)
    out = run(xf)
    return out.reshape(B, C * 8, H // 2, W // 2, Z // 2)


# speedup vs baseline: 2.2402x; 1.5919x over previous
import functools

import jax
import jax.numpy as jnp
from jax import lax
from jax.experimental import pallas as pl
from jax.experimental.pallas import tpu as pltpu
from jax.experimental.pallas import tpu_sc as plsc

_NW = 32
_RPW = 4096 // _NW


def _sc_body(in_hbm, out_hbm, iv0, iv1, ov0, ov1, si0, si1, so0, so1):
    wid = lax.axis_index("s") * 2 + lax.axis_index("c")
    iota2 = lax.iota(jnp.int32, 16) * 2
    ivs = (iv0, iv1)
    ovs = (ov0, ov1)
    sis = (si0, si1)
    sos = (so0, so1)
    cols = [[iota2 + (32 * z16 + k) for z16 in range(4)] for k in range(2)]

    def in_cp(t, b):
        r = wid * _RPW + t
        return pltpu.make_async_copy(
            in_hbm.at[r >> 11, (r >> 7) & 15, r & 127], ivs[b], sis[b]
        )

    def out_cp(t, b, jk):
        r = wid * _RPW + t
        bc = r >> 7
        hr = r & 127
        co = (bc & 15) * 8 + (hr & 1) * 4 + jk
        return pltpu.make_async_copy(
            ovs[b].at[jk], out_hbm.at[bc >> 4, co, hr >> 1], sos[b]
        )

    def compute(b):
        src = ivs[b]
        dst = ovs[b]
        init = (jnp.full((16,), 0, jnp.int32), jnp.full((16,), 1, jnp.int32))

        @plsc.parallel_loop(0, 64, carry=init)
        def w_loop(w, rows):
            for j in range(2):
                for k in range(2):
                    for z16 in range(4):
                        v = plsc.load_gather(src, [rows[j], cols[k][z16]])
                        dst[j * 2 + k, w, pl.ds(z16 * 16, 16)] = v
            return (rows[0] + 2, rows[1] + 2)

    def full_step(t, b):
        in_cp(t, b).wait()
        for jk in range(4):
            out_cp(t, b, jk).wait()
        compute(b)
        in_cp(t + 2, b).start()
        for jk in range(4):
            out_cp(t, b, jk).start()

    in_cp(0, 0).start()
    in_cp(1, 1).start()
    for b in range(2):
        in_cp(b, b).wait()
        compute(b)
        in_cp(b + 2, b).start()
        for jk in range(4):
            out_cp(b, b, jk).start()

    def loop(u, carry):
        full_step(2 * u, 0)
        full_step(2 * u + 1, 1)
        return carry

    lax.fori_loop(1, _RPW // 2 - 1, loop, 0)

    for b in range(2):
        t = _RPW - 2 + b
        in_cp(t, b).wait()
        for jk in range(4):
            out_cp(t, b, jk).wait()
        compute(b)
        for jk in range(4):
            out_cp(t, b, jk).start()
    for b in range(2):
        for jk in range(4):
            out_cp(0, b, jk).wait()


def kernel(x):
    B, C, H, W, Z = x.shape
    mesh = plsc.VectorSubcoreMesh(core_axis_name="c", subcore_axis_name="s")
    run = functools.partial(
        pl.kernel,
        mesh=mesh,
        out_type=jax.ShapeDtypeStruct(
            (B, C * 8, H // 2, W // 2, Z // 2), x.dtype
        ),
        scratch_types=[
            pltpu.VMEM((W, Z), jnp.float32),
            pltpu.VMEM((W, Z), jnp.float32),
            pltpu.VMEM((4, W // 2, Z // 2), jnp.float32),
            pltpu.VMEM((4, W // 2, Z // 2), jnp.float32),
            pltpu.SemaphoreType.DMA,
            pltpu.SemaphoreType.DMA,
            pltpu.SemaphoreType.DMA,
            pltpu.SemaphoreType.DMA,
        ],
        compiler_params=pltpu.CompilerParams(needs_layout_passes=False),
    )(_sc_body)
    return run(x)


# speedup vs baseline: 5.5525x; 2.4786x over previous
import functools

import jax
import jax.numpy as jnp
from jax import lax
from jax.experimental import pallas as pl
from jax.experimental.pallas import tpu as pltpu
from jax.experimental.pallas import tpu_sc as plsc

_UPW = 64


def _sc_body(in_hbm, out_hbm, ib0, ib1, ob0, ob1, si0, si1, so0, so1):
    wid = lax.axis_index("s") * 2 + lax.axis_index("c")
    ibs = (ib0, ib1)
    obs = (ob0, ob1)
    sis = (si0, si1)
    sos = (so0, so1)

    lane = lax.iota(jnp.int32, 16)
    cv = [2 * g + (lane >> 3) for g in range(8)]
    hlv = (lane >> 2) & 1
    jpart = (lane >> 1) & 1
    zz_init = lane & 1

    def decode(u):
        uid = wid * _UPW + u
        return uid >> 10, (uid >> 4) & 63, uid & 15

    def in_cp(u, sb):
        b, h, wq = decode(u)
        return pltpu.make_async_copy(
            in_hbm.at[b, :, pl.ds(2 * h, 2), pl.ds(8 * wq, 8), :],
            ibs[sb],
            sis[sb],
        )

    def out_cp(u, ws, p):
        b, h, wq = decode(u)
        return pltpu.make_async_copy(
            obs[p], out_hbm.at[b, h, wq * 4 + ws], sos[p]
        )

    def compute(u, sb, wait01):
        src = ibs[sb]
        for ws in range(4):
            p = ws & 1
            if ws >= 2 or wait01:
                out_cp(u, ws, p).wait()
            dst = obs[p]
            wrv = 2 * ws + jpart

            @plsc.parallel_loop(0, 64, carry=zz_init, unroll=2)
            def z_loop(z, zzv):
                for g in range(8):
                    v = plsc.load_gather(src, [cv[g], hlv, wrv, zzv])
                    dst[z, pl.ds(16 * g, 16)] = v
                return zzv + 2

            out_cp(u, ws, p).start()

    in_cp(0, 0).start()

    def pair(up, carry):
        for sb in range(2):
            u = 2 * up + sb
            in_cp(u, sb).wait()

            @pl.when(u < _UPW - 1)
            def _():
                in_cp(u + 1, 1 - sb).start()

            if sb == 0:

                @pl.when(up > 0)
                def _():
                    out_cp(u, 0, 0).wait()
                    out_cp(u, 1, 1).wait()

                compute(u, sb, False)
            else:
                compute(u, sb, True)
        return carry

    lax.fori_loop(0, _UPW // 2, pair, 0)

    out_cp(_UPW - 1, 2, 0).wait()
    out_cp(_UPW - 1, 3, 1).wait()


def kernel(x):
    B, C, H, W, Z = x.shape
    mesh = plsc.VectorSubcoreMesh(core_axis_name="c", subcore_axis_name="s")
    run = functools.partial(
        pl.kernel,
        mesh=mesh,
        out_type=jax.ShapeDtypeStruct(
            (B, H // 2, W // 2, Z // 2, C * 8), x.dtype
        ),
        scratch_types=[
            pltpu.VMEM((C, 2, 8, Z), jnp.float32),
            pltpu.VMEM((C, 2, 8, Z), jnp.float32),
            pltpu.VMEM((Z // 2, C * 8), jnp.float32),
            pltpu.VMEM((Z // 2, C * 8), jnp.float32),
            pltpu.SemaphoreType.DMA,
            pltpu.SemaphoreType.DMA,
            pltpu.SemaphoreType.DMA,
            pltpu.SemaphoreType.DMA,
        ],
        compiler_params=pltpu.CompilerParams(needs_layout_passes=False),
    )(_sc_body)
    out = run(x)
    return jnp.transpose(out, (0, 4, 1, 2, 3))
